# baseline (speedup 1.0000x reference)
import jax
import jax.numpy as jnp
from jax.experimental import pallas as pl
from jax.experimental.pallas import tpu as pltpu

_ROWS_PER_BLOCK = 4096


def _matvec_body(x_ref, p_ref, o_ref):
    o_ref[...] = jax.lax.dot_general(
        x_ref[...], p_ref[...],
        dimension_numbers=(((1,), (0,)), ((), ())),
        preferred_element_type=jnp.float32,
    )


def kernel(x, projection):
    b, n, d = x.shape
    rows = b * n
    xf = x.reshape(rows, d)
    grid = rows // _ROWS_PER_BLOCK
    out = pl.pallas_call(
        _matvec_body,
        grid=(grid,),
        in_specs=[
            pl.BlockSpec((_ROWS_PER_BLOCK, d), lambda i: (i, 0)),
            pl.BlockSpec((d, 1), lambda i: (0, 0)),
        ],
        out_specs=pl.BlockSpec((_ROWS_PER_BLOCK, 1), lambda i: (i, 0)),
        out_shape=jax.ShapeDtypeStruct((rows, 1), jnp.float32),
    )(xf, projection)
    return out.reshape(b, n, 1)


# speedup vs baseline: 1.7465x; 1.7465x over previous
import jax
import jax.numpy as jnp
from jax.experimental import pallas as pl
from jax.experimental.pallas import tpu as pltpu

_GROUPS_PER_BLOCK = 32


def _matvec_body(x_ref, p_ref, o_ref):
    y = jax.lax.dot_general(
        x_ref[...], p_ref[...],
        dimension_numbers=(((2,), (0,)), ((), ())),
        preferred_element_type=jnp.float32,
    )
    o_ref[...] = y.reshape(o_ref.shape)


def kernel(x, projection):
    b, n, d = x.shape
    rows = b * n
    groups = rows // d
    xf = x.reshape(groups, d, d)
    grid = groups // _GROUPS_PER_BLOCK
    out = pl.pallas_call(
        _matvec_body,
        grid=(grid,),
        in_specs=[
            pl.BlockSpec((_GROUPS_PER_BLOCK, d, d), lambda i: (i, 0, 0)),
            pl.BlockSpec((d, 1), lambda i: (0, 0)),
        ],
        out_specs=pl.BlockSpec((_GROUPS_PER_BLOCK, d), lambda i: (i, 0)),
        out_shape=jax.ShapeDtypeStruct((groups, d), jnp.float32),
    )(xf, projection)
    return out.reshape(b, n, 1)


# speedup vs baseline: 2.3258x; 1.3317x over previous
import jax
import jax.numpy as jnp
from jax.experimental import pallas as pl
from jax.experimental.pallas import tpu as pltpu

_GROUPS_PER_BLOCK = 64


def _matvec_body(x_ref, p_ref, o_ref):
    y = jax.lax.dot_general(
        x_ref[...], p_ref[...],
        dimension_numbers=(((2,), (0,)), ((), ())),
        preferred_element_type=jnp.float32,
    )
    o_ref[...] = y.reshape(o_ref.shape)


def kernel(x, projection):
    b, n, d = x.shape
    rows = b * n
    groups = rows // d
    xf = x.reshape(groups, d, d)
    grid = groups // _GROUPS_PER_BLOCK
    out = pl.pallas_call(
        _matvec_body,
        grid=(grid,),
        in_specs=[
            pl.BlockSpec((_GROUPS_PER_BLOCK, d, d), lambda i: (i, 0, 0)),
            pl.BlockSpec((d, 1), lambda i: (0, 0)),
        ],
        out_specs=pl.BlockSpec((_GROUPS_PER_BLOCK, d), lambda i: (i, 0)),
        out_shape=jax.ShapeDtypeStruct((groups, d), jnp.float32),
    )(xf, projection)
    return out.reshape(b, n, 1)


# speedup vs baseline: 2.7506x; 1.1826x over previous
import jax
import jax.numpy as jnp
from jax.experimental import pallas as pl
from jax.experimental.pallas import tpu as pltpu

_GROUPS_PER_BLOCK = 128


def _matvec_body(x_ref, p_ref, o_ref):
    y = jax.lax.dot_general(
        x_ref[...], p_ref[...],
        dimension_numbers=(((2,), (0,)), ((), ())),
        preferred_element_type=jnp.float32,
    )
    o_ref[...] = y.reshape(o_ref.shape)


def kernel(x, projection):
    b, n, d = x.shape
    rows = b * n
    groups = rows // d
    xf = x.reshape(groups, d, d)
    grid = groups // _GROUPS_PER_BLOCK
    out = pl.pallas_call(
        _matvec_body,
        grid=(grid,),
        in_specs=[
            pl.BlockSpec((_GROUPS_PER_BLOCK, d, d), lambda i: (i, 0, 0)),
            pl.BlockSpec((d, 1), lambda i: (0, 0)),
        ],
        out_specs=pl.BlockSpec((_GROUPS_PER_BLOCK, d), lambda i: (i, 0)),
        out_shape=jax.ShapeDtypeStruct((groups, d), jnp.float32),
    )(xf, projection)
    return out.reshape(b, n, 1)


# speedup vs baseline: 3.0379x; 1.1044x over previous
import jax
import jax.numpy as jnp
from jax.experimental import pallas as pl
from jax.experimental.pallas import tpu as pltpu

_GROUPS_PER_BLOCK = 256


def _matvec_body(x_ref, p_ref, o_ref):
    y = jax.lax.dot_general(
        x_ref[...], p_ref[...],
        dimension_numbers=(((2,), (0,)), ((), ())),
        preferred_element_type=jnp.float32,
    )
    o_ref[...] = y.reshape(o_ref.shape)


def kernel(x, projection):
    b, n, d = x.shape
    rows = b * n
    groups = rows // d
    xf = x.reshape(groups, d, d)
    grid = groups // _GROUPS_PER_BLOCK
    out = pl.pallas_call(
        _matvec_body,
        grid=(grid,),
        in_specs=[
            pl.BlockSpec((_GROUPS_PER_BLOCK, d, d), lambda i: (i, 0, 0)),
            pl.BlockSpec((d, 1), lambda i: (0, 0)),
        ],
        out_specs=pl.BlockSpec((_GROUPS_PER_BLOCK, d), lambda i: (i, 0)),
        out_shape=jax.ShapeDtypeStruct((groups, d), jnp.float32),
    )(xf, projection)
    return out.reshape(b, n, 1)
